# baseline (speedup 1.0000x reference)
import functools

import jax
import jax.numpy as jnp
from jax import lax
from jax.experimental import pallas as pl
from jax.experimental.pallas import tpu as pltpu
from jax.experimental.pallas import tpu_sc as plsc

NUM_MOVES = 1000
ENTITY_SIZE = 64

NC = 2
NS = 16
NW = NC * NS
CHUNK = 128

NM = 4096 * 4
NH = 4096 * 50
MCH_PER_W = NM // NW // CHUNK
HCH_PER_W = NH // NW // CHUNK


@functools.partial(
    pl.kernel,
    out_type=(
        jax.ShapeDtypeStruct((NM, ENTITY_SIZE), jnp.float32),
        jax.ShapeDtypeStruct((NH, ENTITY_SIZE), jnp.float32),
    ),
    mesh=plsc.VectorSubcoreMesh(core_axis_name="c", subcore_axis_name="s"),
    compiler_params=pltpu.CompilerParams(use_tc_tiling_on_sc=False),
    scratch_types=(
        pltpu.VMEM((NM // NW,), jnp.int32),
        pltpu.VMEM((NH // NW,), jnp.int32),
        pltpu.VMEM((CHUNK, ENTITY_SIZE), jnp.float32),
        pltpu.SemaphoreType.DMA,
    ),
)
def _gather_body(wb_hbm, idxm_hbm, idxh_hbm, outm_hbm, outh_hbm,
                 idxm_v, idxh_v, rows_v, sem):
    wid = lax.axis_index("s") * NC + lax.axis_index("c")

    pltpu.sync_copy(idxm_hbm.at[pl.ds(wid * (NM // NW), NM // NW)], idxm_v)
    pltpu.sync_copy(idxh_hbm.at[pl.ds(wid * (NH // NW), NH // NW)], idxh_v)

    m_base = wid * (NM // NW)
    h_base = wid * (NH // NW)

    def m_step(j, _):
        pltpu.async_copy(
            wb_hbm.at[idxm_v.at[pl.ds(j * CHUNK, CHUNK)]], rows_v, sem).wait()
        pltpu.sync_copy(rows_v, outm_hbm.at[pl.ds(m_base + j * CHUNK, CHUNK)])
        return 0

    def h_step(j, _):
        pltpu.async_copy(
            wb_hbm.at[idxh_v.at[pl.ds(j * CHUNK, CHUNK)]], rows_v, sem).wait()
        pltpu.sync_copy(rows_v, outh_hbm.at[pl.ds(h_base + j * CHUNK, CHUNK)])
        return 0

    lax.fori_loop(0, MCH_PER_W, m_step, 0)
    lax.fori_loop(0, HCH_PER_W, h_step, 0)


def kernel(moveset, history_move_index, W, b):
    wb = W + b[None, :]
    idx_m = moveset[:, :, 0].reshape(NM)
    idx_h = history_move_index.reshape(NH)
    out_m, out_h = _gather_body(wb, idx_m, idx_h)
    return (out_m.reshape(4096, 4, ENTITY_SIZE),
            out_h.reshape(4096, 50, ENTITY_SIZE))


# speedup vs baseline: 1.0341x; 1.0341x over previous
import functools

import jax
import jax.numpy as jnp
from jax import lax
from jax.experimental import pallas as pl
from jax.experimental.pallas import tpu as pltpu
from jax.experimental.pallas import tpu_sc as plsc

NUM_MOVES = 1000
ENTITY_SIZE = 64

NC = 2
NS = 16
NW = NC * NS
CHUNK = 128

NM = 4096 * 4
NH = 4096 * 50
M_PER_W = NM // NW
H_PER_W = NH // NW
SUP = 5 * CHUNK


@functools.partial(
    pl.kernel,
    out_type=(
        jax.ShapeDtypeStruct((NM, ENTITY_SIZE), jnp.float32),
        jax.ShapeDtypeStruct((NH, ENTITY_SIZE), jnp.float32),
    ),
    mesh=plsc.VectorSubcoreMesh(core_axis_name="c", subcore_axis_name="s"),
    compiler_params=pltpu.CompilerParams(use_tc_tiling_on_sc=False),
    scratch_types=(
        pltpu.VMEM((M_PER_W,), jnp.int32),
        pltpu.VMEM((H_PER_W,), jnp.int32),
        pltpu.VMEM((SUP, ENTITY_SIZE), jnp.float32),
        pltpu.VMEM((SUP, ENTITY_SIZE), jnp.float32),
        pltpu.SemaphoreType.DMA,
        pltpu.SemaphoreType.DMA,
        pltpu.SemaphoreType.DMA,
        pltpu.SemaphoreType.DMA,
    ),
)
def _gather_body(wb_hbm, idxm_hbm, idxh_hbm, outm_hbm, outh_hbm,
                 idxm_v, idxh_v, rows0, rows1, gsem0, gsem1, wsem0, wsem1):
    wid = lax.axis_index("s") * NC + lax.axis_index("c")

    pltpu.sync_copy(idxm_hbm.at[pl.ds(wid * M_PER_W, M_PER_W)], idxm_v)
    pltpu.sync_copy(idxh_hbm.at[pl.ds(wid * H_PER_W, H_PER_W)], idxh_v)

    rows = (rows0, rows1)
    gsem = (gsem0, gsem1)
    wsem = (wsem0, wsem1)
    pending_write = [None, None]

    def run_super(b, idx_v, idx_off, nrows, out_hbm, out_off):
        if pending_write[b] is not None:
            pending_write[b].wait()
        gathers = []
        for k in range(nrows // CHUNK):
            gathers.append(pltpu.async_copy(
                wb_hbm.at[idx_v.at[pl.ds(idx_off + k * CHUNK, CHUNK)]],
                rows[b].at[pl.ds(k * CHUNK, CHUNK)],
                gsem[b]))
        for g in gathers:
            g.wait()
        pending_write[b] = pltpu.async_copy(
            rows[b].at[pl.ds(0, nrows)],
            out_hbm.at[pl.ds(out_off, nrows)],
            wsem[b])

    m_base = wid * M_PER_W
    run_super(0, idxm_v, 0, M_PER_W // 2, outm_hbm, m_base)
    run_super(1, idxm_v, M_PER_W // 2, M_PER_W // 2, outm_hbm,
              m_base + M_PER_W // 2)

    h_base = wid * H_PER_W
    for s in range(H_PER_W // SUP):
        run_super(s % 2, idxh_v, s * SUP, SUP, outh_hbm, h_base + s * SUP)

    pending_write[0].wait()
    pending_write[1].wait()


def kernel(moveset, history_move_index, W, b):
    wb = W + b[None, :]
    idx_m = moveset[:, :, 0].reshape(NM)
    idx_h = history_move_index.reshape(NH)
    out_m, out_h = _gather_body(wb, idx_m, idx_h)
    return (out_m.reshape(4096, 4, ENTITY_SIZE),
            out_h.reshape(4096, 50, ENTITY_SIZE))


# speedup vs baseline: 1.2996x; 1.2567x over previous
import functools

import jax
import jax.numpy as jnp
from jax import lax
from jax.experimental import pallas as pl
from jax.experimental.pallas import tpu as pltpu
from jax.experimental.pallas import tpu_sc as plsc

NUM_MOVES = 1000
ENTITY_SIZE = 64
D = ENTITY_SIZE
TPAD = 1024
B = 4096
JM = 4
JH = 50

NC = 2
NS = 16
NW = NC * NS
LPW = B // NW
L = 16
NG = LPW // L


@functools.partial(
    pl.kernel,
    out_type=(
        jax.ShapeDtypeStruct((JM, D, B), jnp.float32),
        jax.ShapeDtypeStruct((JH, D, B), jnp.float32),
    ),
    mesh=plsc.VectorSubcoreMesh(core_axis_name="c", subcore_axis_name="s"),
    compiler_params=pltpu.CompilerParams(needs_layout_passes=False),
    scratch_types=(
        pltpu.VMEM((D * TPAD,), jnp.float32),
        pltpu.VMEM((JM, LPW), jnp.int32),
        pltpu.VMEM((JH, LPW), jnp.int32),
        pltpu.VMEM((2, D, LPW), jnp.float32),
        pltpu.SemaphoreType.DMA,
    ),
)
def _enc_body(wt_hbm, idxm_hbm, idxh_hbm, outm_hbm, outh_hbm,
              table_v, idxm_v, idxh_v, buf, wsem):
    wid = lax.axis_index("s") * NC + lax.axis_index("c")
    lane0 = wid * LPW

    pltpu.sync_copy(wt_hbm, table_v)
    pltpu.sync_copy(idxm_hbm.at[:, pl.ds(lane0, LPW)], idxm_v)
    pltpu.sync_copy(idxh_hbm.at[:, pl.ds(lane0, LPW)], idxh_v)

    def make_tile(j, idx_v, bsel):
        for g in range(NG):
            addr = idx_v[j, pl.ds(g * L, L)]
            for c in range(D):
                buf[bsel, c, pl.ds(g * L, L)] = plsc.load_gather(
                    table_v, [addr])
                addr = addr + TPAD

    def drain():
        pltpu.make_async_copy(
            buf.at[0], outh_hbm.at[0, :, pl.ds(0, LPW)], wsem).wait()

    def run(nj, idx_v, out_hbm):
        def step(j, _):
            bsel = lax.rem(j, 2)

            @pl.when(j >= 2)
            def _():
                drain()

            make_tile(j, idx_v, bsel)
            pltpu.async_copy(
                buf.at[bsel], out_hbm.at[j, :, pl.ds(lane0, LPW)], wsem)
            return 0

        lax.fori_loop(0, nj, step, 0)
        drain()
        drain()

    run(JM, idxm_v, outm_hbm)
    run(JH, idxh_v, outh_hbm)


def kernel(moveset, history_move_index, W, b):
    wt = (W + b[None, :]).T
    wt_flat = jnp.pad(wt, ((0, 0), (0, TPAD - NUM_MOVES))).reshape(D * TPAD)
    idx_m = moveset[:, :, 0].T
    idx_h = history_move_index.T
    o_m, o_h = _enc_body(wt_flat, idx_m, idx_h)
    return (o_m.transpose(2, 0, 1), o_h.transpose(2, 0, 1))


# speedup vs baseline: 1.3045x; 1.0038x over previous
import functools

import jax
import jax.numpy as jnp
from jax import lax
from jax.experimental import pallas as pl
from jax.experimental.pallas import tpu as pltpu
from jax.experimental.pallas import tpu_sc as plsc

NUM_MOVES = 1000
ENTITY_SIZE = 64
D = ENTITY_SIZE
TPAD = 1024
B = 4096
JM = 4
JH = 50

NC = 2
NS = 16
NW = NC * NS
LPW = B // NW
L = 16
NG = LPW // L


@functools.partial(
    pl.kernel,
    out_type=(
        jax.ShapeDtypeStruct((JM, D, B), jnp.float32),
        jax.ShapeDtypeStruct((JH, D, B), jnp.float32),
    ),
    mesh=plsc.VectorSubcoreMesh(core_axis_name="c", subcore_axis_name="s"),
    compiler_params=pltpu.CompilerParams(needs_layout_passes=False),
    scratch_types=(
        pltpu.VMEM((D * TPAD,), jnp.float32),
        pltpu.VMEM((JM, LPW), jnp.int32),
        pltpu.VMEM((JH, LPW), jnp.int32),
        pltpu.VMEM((2, D, LPW), jnp.float32),
        pltpu.SemaphoreType.DMA,
    ),
)
def _enc_body(wt_hbm, idxm_hbm, idxh_hbm, outm_hbm, outh_hbm,
              table_v, idxm_v, idxh_v, buf, wsem):
    wid = lax.axis_index("s") * NC + lax.axis_index("c")
    lane0 = wid * LPW

    pltpu.sync_copy(wt_hbm, table_v)
    pltpu.sync_copy(idxm_hbm.at[:, pl.ds(lane0, LPW)], idxm_v)
    pltpu.sync_copy(idxh_hbm.at[:, pl.ds(lane0, LPW)], idxh_v)

    def make_tile(j, idx_v, bsel):
        for g in range(NG):
            base = idx_v[j, pl.ds(g * L, L)]
            for c in range(D):
                buf[bsel, c, pl.ds(g * L, L)] = plsc.load_gather(
                    table_v, [base + c * TPAD])

    def drain():
        pltpu.make_async_copy(
            buf.at[0], outh_hbm.at[0, :, pl.ds(0, LPW)], wsem).wait()

    def run(nj, idx_v, out_hbm):
        def step(j, _):
            bsel = lax.rem(j, 2)

            @pl.when(j >= 2)
            def _():
                drain()

            make_tile(j, idx_v, bsel)
            pltpu.async_copy(
                buf.at[bsel], out_hbm.at[j, :, pl.ds(lane0, LPW)], wsem)
            return 0

        lax.fori_loop(0, nj, step, 0)
        drain()
        drain()

    run(JM, idxm_v, outm_hbm)
    run(JH, idxh_v, outh_hbm)


def kernel(moveset, history_move_index, W, b):
    wt = (W + b[None, :]).T
    wt_flat = jnp.pad(wt, ((0, 0), (0, TPAD - NUM_MOVES))).reshape(D * TPAD)
    idx_m = moveset[:, :, 0].T
    idx_h = history_move_index.T
    o_m, o_h = _enc_body(wt_flat, idx_m, idx_h)
    return (o_m.transpose(2, 0, 1), o_h.transpose(2, 0, 1))


# speedup vs baseline: 4.3234x; 3.3142x over previous
import functools

import jax
import jax.numpy as jnp
from jax import lax
from jax.experimental import pallas as pl
from jax.experimental.pallas import tpu as pltpu
from jax.experimental.pallas import tpu_sc as plsc

NUM_MOVES = 1000
ENTITY_SIZE = 64
D = ENTITY_SIZE
TPAD = 1024
B = 4096
JM = 4
JH = 50

NC = 2
NS = 16
NW = NC * NS
LPW = B // NW
L = 16
NG = LPW // L
NSLAB = 4
CSLAB = D // NSLAB


@functools.partial(
    pl.kernel,
    out_type=(
        jax.ShapeDtypeStruct((JM, D, B), jnp.float32),
        jax.ShapeDtypeStruct((JH, D, B), jnp.float32),
    ),
    mesh=plsc.VectorSubcoreMesh(core_axis_name="c", subcore_axis_name="s"),
    compiler_params=pltpu.CompilerParams(needs_layout_passes=False),
    scratch_types=(
        pltpu.VMEM((D * TPAD,), jnp.float32),
        pltpu.VMEM((JM, LPW), jnp.int32),
        pltpu.VMEM((JH, LPW), jnp.int32),
        pltpu.VMEM((4, CSLAB, LPW), jnp.float32),
        pltpu.SemaphoreType.DMA,
        pltpu.SemaphoreType.DMA,
        pltpu.SemaphoreType.DMA,
    ),
)
def _enc_body(wt_hbm, idxm_hbm, idxh_hbm, outm_hbm, outh_hbm,
              table_v, idxm_v, idxh_v, buf, wsem, ssem, isem):
    wid = lax.axis_index("s") * NC + lax.axis_index("c")
    lane0 = wid * LPW
    slab_words = CSLAB * TPAD

    for k in range(NSLAB):
        pltpu.async_copy(wt_hbm.at[pl.ds(k * slab_words, slab_words)],
                         table_v.at[pl.ds(k * slab_words, slab_words)], ssem)
    pltpu.async_copy(idxm_hbm.at[:, pl.ds(lane0, LPW)], idxm_v, isem)
    pltpu.async_copy(idxh_hbm.at[:, pl.ds(lane0, LPW)], idxh_v, isem)

    def wait_slab():
        pltpu.make_async_copy(wt_hbm.at[pl.ds(0, slab_words)],
                              table_v.at[pl.ds(0, slab_words)], ssem).wait()

    def wait_idx():
        pltpu.make_async_copy(idxm_hbm.at[:, pl.ds(0, LPW)], idxm_v,
                              isem).wait()
        pltpu.make_async_copy(idxh_hbm.at[:, pl.ds(0, LPW)], idxh_v,
                              isem).wait()

    def make_block(j, idx_v, bsel, c0):
        bases = [idx_v[j, pl.ds(g * L, L)] for g in range(NG)]

        def load_row(c):
            return [plsc.load_gather(table_v, [bases[g] + c * TPAD])
                    for g in range(NG)]

        prev = load_row(c0)
        for cc in range(1, CSLAB):
            cur = []
            for g in range(NG):
                cur.append(plsc.load_gather(table_v,
                                            [bases[g] + (c0 + cc) * TPAD]))
                buf[bsel, cc - 1, pl.ds(g * L, L)] = prev[g]
            prev = cur
        for g in range(NG):
            buf[bsel, CSLAB - 1, pl.ds(g * L, L)] = prev[g]

    def drain():
        pltpu.make_async_copy(
            buf.at[0], outh_hbm.at[0, pl.ds(0, CSLAB), pl.ds(0, LPW)],
            wsem).wait()

    def run(nj, idx_v, out_hbm, c0, off, drain_always):
        def step(j, _):
            bsel = lax.rem(j + off, 4)

            if drain_always:
                drain()
            else:
                @pl.when(j >= 4)
                def _():
                    drain()

            make_block(j, idx_v, bsel, c0)
            pltpu.async_copy(
                buf.at[bsel],
                out_hbm.at[j, pl.ds(c0, CSLAB), pl.ds(lane0, LPW)], wsem)
            return 0

        lax.fori_loop(0, nj, step, 0)

    wait_idx()
    nsteps = 0
    for k in range(NSLAB):
        wait_slab()
        c0 = k * CSLAB
        run(JM, idxm_v, outm_hbm, c0, nsteps % 4, drain_always=(k > 0))
        nsteps += JM
        run(JH, idxh_v, outh_hbm, c0, nsteps % 4, drain_always=True)
        nsteps += JH
    for _ in range(4):
        drain()


def kernel(moveset, history_move_index, W, b):
    wt = (W + b[None, :]).T
    wt_flat = jnp.pad(wt, ((0, 0), (0, TPAD - NUM_MOVES))).reshape(D * TPAD)
    idx_m = moveset[:, :, 0].T
    idx_h = history_move_index.T
    o_m, o_h = _enc_body(wt_flat, idx_m, idx_h)
    return (o_m.transpose(2, 0, 1), o_h.transpose(2, 0, 1))
